# Initial kernel scaffold; baseline (speedup 1.0000x reference)
import functools

import jax
import jax.numpy as jnp
from jax import lax
from jax.experimental import pallas as pl
from jax.experimental.pallas import tpu as pltpu
from jax.experimental.pallas import tpu_sc as plsc

N = 10000
E = 320000
D = 128
G = 64
C = 16

NTILES = 16
EPT = 20480
NB = EPT // 128
NP = N + 128
RPT = NP // NTILES
RCH = 211

_MESH = plsc.VectorSubcoreMesh(core_axis_name="c", subcore_axis_name="s")


@functools.partial(
    pl.kernel,
    out_type=jax.ShapeDtypeStruct((2, NP, 8), jnp.float32),
    mesh=_MESH,
    scratch_types=[
        pltpu.VMEM_SHARED((NP, 8), jnp.float32),
        pltpu.VMEM((NB, 128), jnp.int32),
        pltpu.VMEM((128, 8), jnp.float32),
        pltpu.VMEM((RPT, 8), jnp.float32),
    ],
)
def _deg_kernel(dstb_hbm, ones_hbm, zeros_hbm, out_hbm, dacc, dst_v, ones_v, zbuf):
    c = lax.axis_index("c")
    s = lax.axis_index("s")
    w = c * NTILES + s
    pltpu.sync_copy(zeros_hbm, zbuf)
    pltpu.sync_copy(zbuf, dacc.at[pl.ds(s * RPT, RPT)])
    pltpu.sync_copy(ones_hbm, ones_v)
    pltpu.sync_copy(dstb_hbm.at[w], dst_v)
    plsc.subcore_barrier()

    def step(j, carry):
        pltpu.sync_copy(ones_v, dacc.at[dst_v.at[j]], add=True)
        return carry

    lax.fori_loop(0, NB, step, 0)
    plsc.subcore_barrier()
    pltpu.sync_copy(dacc.at[pl.ds(s * RPT, RPT)], zbuf)
    pltpu.sync_copy(zbuf, out_hbm.at[c].at[pl.ds(s * RPT, RPT)])


@functools.partial(
    pl.kernel,
    out_type=jax.ShapeDtypeStruct((2, NP, D), jnp.float32),
    mesh=_MESH,
    scratch_types=[
        pltpu.VMEM_SHARED((NP, D), jnp.float32),
        pltpu.VMEM((NB, 128), jnp.int32),
        pltpu.VMEM((NB, 128), jnp.int32),
        pltpu.VMEM((2, 128, D), jnp.float32),
        pltpu.VMEM((RCH, D), jnp.float32),
        pltpu.SemaphoreType.DMA,
        pltpu.SemaphoreType.DMA,
    ],
)
def _conv_kernel(h_hbm, srcb_hbm, dstb_hbm, out_hbm,
                 acc, src_v, dst_v, msg, obuf, sem0, sem1):
    c = lax.axis_index("c")
    s = lax.axis_index("s")
    w = c * NTILES + s

    def zrow(i, carry):
        for j in range(D // 16):
            obuf[i, pl.ds(j * 16, 16)] = jnp.zeros((16,), jnp.float32)
        return carry

    lax.fori_loop(0, RCH, zrow, 0)
    row0 = s * RPT
    for k in range(3):
        pltpu.sync_copy(obuf, acc.at[pl.ds(row0 + k * RCH, RCH)])

    pltpu.sync_copy(srcb_hbm.at[w], src_v)
    pltpu.sync_copy(dstb_hbm.at[w], dst_v)
    plsc.subcore_barrier()

    pltpu.async_copy(h_hbm.at[src_v.at[0]], msg.at[0], sem0)

    def step(k, carry):
        j0 = 2 * k
        j1 = j0 + 1
        pltpu.async_copy(h_hbm.at[src_v.at[j1]], msg.at[1], sem1)
        pltpu.make_async_copy(h_hbm.at[src_v.at[j0]], msg.at[0], sem0).wait()
        pltpu.sync_copy(msg.at[0], acc.at[dst_v.at[j0]], add=True)

        @pl.when(j1 + 1 < NB)
        def _():
            pltpu.async_copy(h_hbm.at[src_v.at[j1 + 1]], msg.at[0], sem0)

        pltpu.make_async_copy(h_hbm.at[src_v.at[j1]], msg.at[1], sem1).wait()
        pltpu.sync_copy(msg.at[1], acc.at[dst_v.at[j1]], add=True)
        return carry

    lax.fori_loop(0, NB // 2, step, 0)
    plsc.subcore_barrier()

    for k in range(3):
        r0 = row0 + k * RCH
        pltpu.sync_copy(acc.at[pl.ds(r0, RCH)], obuf)
        pltpu.sync_copy(obuf, out_hbm.at[c].at[pl.ds(r0, RCH)])


def _dinv(deg8):
    deg = deg8[:, 0:1] + 1.0
    return lax.rsqrt(jnp.maximum(deg, 1e-12))


def _mm1_body(x_ref, w_ref, deg_ref, o_ref):
    h = jnp.dot(x_ref[...], w_ref[...], preferred_element_type=jnp.float32)
    o_ref[...] = h * _dinv(deg_ref[...])


def _mid_body(acc_ref, h1p_ref, deg_ref, w2_ref, b1_ref, o_ref):
    dinv = _dinv(deg_ref[...])
    z = jnp.maximum(dinv * (acc_ref[...] + h1p_ref[...]) + b1_ref[...], 0.0)
    o_ref[...] = jnp.dot(z, w2_ref[...], preferred_element_type=jnp.float32) * dinv


def _tail_body(acc_ref, h2p_ref, deg_ref, seg_ref, lab_ref, b2_ref,
               wc1_ref, bc1_ref, wc2_ref, bc2_ref, wd_ref, bd_ref,
               o1_ref, o2_ref, o3_ref):
    dinv = _dinv(deg_ref[...])
    x = dinv * (acc_ref[...] + h2p_ref[...]) + b2_ref[...]
    seg = seg_ref[...]
    oh = (seg == lax.broadcasted_iota(jnp.float32, (2 * N, 2 * G), 1))
    oh = oh.astype(jnp.float32)
    cdims = (((0,), (0,)), ((), ()))
    pooled = lax.dot_general(oh, x, cdims,
                             preferred_element_type=jnp.float32)
    cnt = lax.dot_general(oh, jnp.ones((2 * N, 2 * G), jnp.float32), cdims,
                          preferred_element_type=jnp.float32)
    xb = pooled / jnp.maximum(cnt, 1.0)
    xds = xb[0:G, :]
    xdt = xb[G:2 * G, :]
    hh = jnp.maximum(
        jnp.dot(xds, wc1_ref[...], preferred_element_type=jnp.float32)
        + bc1_ref[...], 0.0)
    p = jax.nn.sigmoid(
        jnp.dot(hh, wc2_ref[...], preferred_element_type=jnp.float32)
        + bc2_ref[...])
    p = jnp.clip(p, 1e-7, 1.0 - 1e-7)
    y = (lab_ref[...] == lax.broadcasted_iota(jnp.float32, (G, C), 1))
    y = y.astype(jnp.float32)
    clf = -jnp.mean(y * jnp.log(p) + (1.0 - y) * jnp.log(1.0 - p))
    cr = jax.nn.sigmoid(
        jnp.dot(xb, wd_ref[...], preferred_element_type=jnp.float32)
        + bd_ref[...])
    m_ds = jnp.mean(cr[0:G, :])
    m_dt = jnp.mean(cr[G:2 * G, :])
    dl = jnp.abs(m_ds - m_dt)
    o1_ref[0, 0] = clf + dl
    o2_ref[0, 0] = clf
    o3_ref[0, 0] = dl


_mm1 = pl.pallas_call(
    _mm1_body,
    out_shape=jax.ShapeDtypeStruct((2 * N, D), jnp.float32),
)

_mid = pl.pallas_call(
    _mid_body,
    out_shape=jax.ShapeDtypeStruct((2 * N, D), jnp.float32),
)

_tail = pl.pallas_call(
    _tail_body,
    out_shape=(
        jax.ShapeDtypeStruct((1, 1), jnp.float32),
        jax.ShapeDtypeStruct((1, 1), jnp.float32),
        jax.ShapeDtypeStruct((1, 1), jnp.float32),
    ),
)


def _edge_blocks(src, dst, goff):
    pad = EPT * NTILES - E
    ar = jnp.arange(pad, dtype=jnp.int32)
    pad_src = (ar * 67) % N + goff
    pad_dst = N + (ar % 128)
    s_all = jnp.concatenate([src + goff, pad_src]).reshape(NTILES, NB, 128)
    d_all = jnp.concatenate([dst, pad_dst]).reshape(NTILES, NB, 128)
    return s_all, d_all


def kernel(features_s, edge_index_s, batch_s, labels_s,
           features_t, edge_index_t, batch_t,
           W1, b1, W2, b2, Wc1, bc1, Wc2, bc2, Wd, bd):
    x = jnp.concatenate([features_s, features_t], axis=0)

    ss, ds_ = _edge_blocks(edge_index_s[0], edge_index_s[1], 0)
    st, dt_ = _edge_blocks(edge_index_t[0], edge_index_t[1], N)
    srcb = jnp.concatenate([ss[None], st[None]]).reshape(2 * NTILES, NB, 128)
    dstb = jnp.concatenate([ds_[None], dt_[None]]).reshape(2 * NTILES, NB, 128)

    ones8 = jnp.ones((128, 8), jnp.float32)
    zeros8 = jnp.zeros((RPT, 8), jnp.float32)
    dout = _deg_kernel(dstb, ones8, zeros8)
    deg8 = dout[:, :N, :].reshape(2 * N, 8)

    h1p = _mm1(x, W1, deg8)
    acc1 = _conv_kernel(h1p, srcb, dstb)[:, :N, :].reshape(2 * N, D)
    h2p = _mid(acc1, h1p, deg8, W2, b1.reshape(1, D))
    acc2 = _conv_kernel(h2p, srcb, dstb)[:, :N, :].reshape(2 * N, D)

    seg = jnp.concatenate([batch_s, batch_t + G]).astype(jnp.float32)
    seg = seg.reshape(2 * N, 1)
    lab = labels_s.astype(jnp.float32).reshape(G, 1)

    tot, clf, dl = _tail(acc2, h2p, deg8, seg, lab, b2.reshape(1, D),
                         Wc1, bc1.reshape(1, C), Wc2, bc2.reshape(1, C),
                         Wd, bd.reshape(1, 1))
    return tot.reshape(()), clf.reshape(()), dl.reshape(())


# speedup vs baseline: 22.5009x; 22.5009x over previous
import functools

import jax
import jax.numpy as jnp
from jax import lax
from jax.experimental import pallas as pl
from jax.experimental.pallas import tpu as pltpu
from jax.experimental.pallas import tpu_sc as plsc

N = 10000
E = 320000
D = 128
G = 64
C = 16

NTILES = 16
EPT = 20480
NB = EPT // 128
NP = 10240
RPT = NP // NTILES
RCH = 160
NCH = RPT // RCH

def _deg_body(dstb_hbm, ones_hbm, out_hbm, dacc, didx, ones_v, zbuf, semi):
    c = lax.axis_index("c")
    s = lax.axis_index("s")
    w = c * NTILES + s

    def zrow(i, carry):
        for j in range(D // 16):
            zbuf[i, pl.ds(j * 16, 16)] = jnp.zeros((16,), jnp.float32)
        return carry

    lax.fori_loop(0, 128, zrow, 0)
    row0 = s * RPT
    for k in range(RPT // 128):
        pltpu.sync_copy(zbuf, dacc.at[pl.ds(row0 + k * 128, 128)])
    pltpu.sync_copy(ones_hbm, ones_v)
    plsc.subcore_barrier()

    pltpu.async_copy(dstb_hbm.at[w].at[pl.ds(0, CB)], didx.at[0], semi)
    for ch in range(NCHK):
        b = ch % 2
        pltpu.make_async_copy(
            dstb_hbm.at[w].at[pl.ds(ch * CB, CB)], didx.at[b], semi).wait()
        if ch + 1 < NCHK:
            pltpu.async_copy(dstb_hbm.at[w].at[pl.ds((ch + 1) * CB, CB)],
                             didx.at[1 - b], semi)
        dv = didx.at[b]

        def step(j, carry, dv=dv):
            pltpu.sync_copy(ones_v, dacc.at[dv.at[j]], add=True)
            return carry

        lax.fori_loop(0, CB, step, 0)

    plsc.subcore_barrier()
    for k in range(RPT // 128):
        r0 = row0 + k * 128
        pltpu.sync_copy(dacc.at[pl.ds(r0, 128)], zbuf)
        pltpu.sync_copy(zbuf, out_hbm.at[c].at[pl.ds(r0, 128)])


CB = 8
NCHK = NB // CB


def _conv_body(h_hbm, srcb_hbm, dstb_hbm, out_hbm,
               acc, sidx, didx, msg, semi, sem0, sem1):
    c = lax.axis_index("c")
    s = lax.axis_index("s")
    w = c * NTILES + s

    def zrow(i, carry):
        for j in range(D // 16):
            msg[0, i, pl.ds(j * 16, 16)] = jnp.zeros((16,), jnp.float32)
        return carry

    lax.fori_loop(0, 128, zrow, 0)
    row0 = s * RPT
    for k in range(RPT // 128):
        pltpu.sync_copy(msg.at[0], acc.at[pl.ds(row0 + k * 128, 128)])
    plsc.subcore_barrier()

    pltpu.async_copy(srcb_hbm.at[w].at[pl.ds(0, CB)], sidx.at[0], semi)
    pltpu.async_copy(dstb_hbm.at[w].at[pl.ds(0, CB)], didx.at[0], semi)

    for ch in range(NCHK):
        b = ch % 2
        pltpu.make_async_copy(
            srcb_hbm.at[w].at[pl.ds(ch * CB, CB)], sidx.at[b], semi).wait()
        pltpu.make_async_copy(
            dstb_hbm.at[w].at[pl.ds(ch * CB, CB)], didx.at[b], semi).wait()
        if ch + 1 < NCHK:
            o0 = (ch + 1) * CB
            pltpu.async_copy(srcb_hbm.at[w].at[pl.ds(o0, CB)],
                             sidx.at[1 - b], semi)
            pltpu.async_copy(dstb_hbm.at[w].at[pl.ds(o0, CB)],
                             didx.at[1 - b], semi)
        sv = sidx.at[b]
        dv = didx.at[b]

        pltpu.async_copy(h_hbm.at[sv.at[0]], msg.at[0], sem0)

        def istep(k2, carry, sv=sv, dv=dv):
            j0 = 2 * k2
            j1 = j0 + 1
            pltpu.async_copy(h_hbm.at[sv.at[j1]], msg.at[1], sem1)
            pltpu.make_async_copy(h_hbm.at[sv.at[j0]], msg.at[0], sem0).wait()
            pltpu.sync_copy(msg.at[0], acc.at[dv.at[j0]], add=True)

            @pl.when(j1 + 1 < CB)
            def _():
                pltpu.async_copy(h_hbm.at[sv.at[j1 + 1]], msg.at[0], sem0)

            pltpu.make_async_copy(h_hbm.at[sv.at[j1]], msg.at[1], sem1).wait()
            pltpu.sync_copy(msg.at[1], acc.at[dv.at[j1]], add=True)
            return carry

        lax.fori_loop(0, CB // 2, istep, 0)

    plsc.subcore_barrier()

    for k in range(RPT // 128):
        r0 = row0 + k * 128
        pltpu.sync_copy(acc.at[pl.ds(r0, 128)], msg.at[0])
        pltpu.sync_copy(msg.at[0], out_hbm.at[c].at[pl.ds(r0, 128)])


@functools.lru_cache(maxsize=None)
def _sc_kernels():
    mesh = plsc.VectorSubcoreMesh(core_axis_name="c", subcore_axis_name="s")
    deg = pl.kernel(
        _deg_body,
        out_type=jax.ShapeDtypeStruct((2, NP, D), jnp.float32),
        mesh=mesh,
        scratch_types=[
            pltpu.VMEM_SHARED((NP, D), jnp.float32),
            pltpu.VMEM((2, CB, 128), jnp.int32),
            pltpu.VMEM((128, D), jnp.float32),
            pltpu.VMEM((128, D), jnp.float32),
            pltpu.SemaphoreType.DMA,
        ],
    )
    conv = pl.kernel(
        _conv_body,
        out_type=jax.ShapeDtypeStruct((2, NP, D), jnp.float32),
        mesh=mesh,
        scratch_types=[
            pltpu.VMEM_SHARED((NP, D), jnp.float32),
            pltpu.VMEM((2, CB, 128), jnp.int32),
            pltpu.VMEM((2, CB, 128), jnp.int32),
            pltpu.VMEM((2, 128, D), jnp.float32),
            pltpu.SemaphoreType.DMA,
            pltpu.SemaphoreType.DMA,
            pltpu.SemaphoreType.DMA,
        ],
    )
    return deg, conv


def _dinv(deg8):
    deg = deg8[:, 0:1] + 1.0
    return lax.rsqrt(jnp.maximum(deg, 1e-12))


def _mm1_body(x_ref, w_ref, deg_ref, o_ref):
    h = jnp.dot(x_ref[...], w_ref[...], preferred_element_type=jnp.float32)
    o_ref[...] = h * _dinv(deg_ref[...])


def _mid_body(acc_ref, h1p_ref, deg_ref, w2_ref, b1_ref, o_ref):
    dinv = _dinv(deg_ref[...])
    z = jnp.maximum(dinv * (acc_ref[...] + h1p_ref[...]) + b1_ref[...], 0.0)
    o_ref[...] = jnp.dot(z, w2_ref[...], preferred_element_type=jnp.float32) * dinv


def _tail_body(acc_ref, h2p_ref, deg_ref, seg_ref, lab_ref, b2_ref,
               wc1_ref, bc1_ref, wc2_ref, bc2_ref, wd_ref, bd_ref,
               o1_ref, o2_ref, o3_ref):
    dinv = _dinv(deg_ref[...])
    x = dinv * (acc_ref[...] + h2p_ref[...]) + b2_ref[...]
    seg = seg_ref[...]
    oh = (seg == lax.broadcasted_iota(jnp.int32, (2 * N, 2 * G), 1))
    oh = oh.astype(jnp.float32)
    cdims = (((0,), (0,)), ((), ()))
    pooled = lax.dot_general(oh, x, cdims,
                             preferred_element_type=jnp.float32)
    cnt = lax.dot_general(oh, jnp.ones((2 * N, 2 * G), jnp.float32), cdims,
                          preferred_element_type=jnp.float32)
    xb = pooled / jnp.maximum(cnt, 1.0)
    xds = xb[0:G, :]
    xdt = xb[G:2 * G, :]
    hh = jnp.maximum(
        jnp.dot(xds, wc1_ref[...], preferred_element_type=jnp.float32)
        + bc1_ref[...], 0.0)
    p = jax.nn.sigmoid(
        jnp.dot(hh, wc2_ref[...], preferred_element_type=jnp.float32)
        + bc2_ref[...])
    p = jnp.clip(p, 1e-7, 1.0 - 1e-7)
    y = (lab_ref[...] == lax.broadcasted_iota(jnp.int32, (G, C), 1))
    y = y.astype(jnp.float32)
    clf = -jnp.mean(y * jnp.log(p) + (1.0 - y) * jnp.log(1.0 - p))
    cr = jax.nn.sigmoid(
        jnp.dot(xb, wd_ref[...], preferred_element_type=jnp.float32)
        + bd_ref[...])
    m_ds = jnp.mean(cr[0:G, :])
    m_dt = jnp.mean(cr[G:2 * G, :])
    dl = jnp.abs(m_ds - m_dt)
    o1_ref[...] = (clf + dl).reshape(1, 1)
    o2_ref[...] = clf.reshape(1, 1)
    o3_ref[...] = dl.reshape(1, 1)


_mm1 = pl.pallas_call(
    _mm1_body,
    out_shape=jax.ShapeDtypeStruct((2 * N, D), jnp.float32),
)

_mid = pl.pallas_call(
    _mid_body,
    out_shape=jax.ShapeDtypeStruct((2 * N, D), jnp.float32),
)

_tail = pl.pallas_call(
    _tail_body,
    out_shape=(
        jax.ShapeDtypeStruct((1, 1), jnp.float32),
        jax.ShapeDtypeStruct((1, 1), jnp.float32),
        jax.ShapeDtypeStruct((1, 1), jnp.float32),
    ),
)


def _edge_blocks(src, dst, goff):
    pad = EPT * NTILES - E
    ar = jnp.arange(pad, dtype=jnp.int32)
    pad_src = (ar * 67) % N + goff
    pad_dst = N + (ar % 128)
    s_all = jnp.concatenate([src + goff, pad_src]).reshape(NTILES, NB, 128)
    d_all = jnp.concatenate([dst, pad_dst]).reshape(NTILES, NB, 128)
    return s_all, d_all


def kernel(features_s, edge_index_s, batch_s, labels_s,
           features_t, edge_index_t, batch_t,
           W1, b1, W2, b2, Wc1, bc1, Wc2, bc2, Wd, bd):
    x = jnp.concatenate([features_s, features_t], axis=0)

    ss, ds_ = _edge_blocks(edge_index_s[0], edge_index_s[1], 0)
    st, dt_ = _edge_blocks(edge_index_t[0], edge_index_t[1], N)
    srcb = jnp.concatenate([ss[None], st[None]]).reshape(2 * NTILES, NB, 128)
    dstb = jnp.concatenate([ds_[None], dt_[None]]).reshape(2 * NTILES, NB, 128)

    _deg_kernel, _conv_kernel = _sc_kernels()
    ones128 = jnp.ones((128, D), jnp.float32)
    dout = _deg_kernel(dstb, ones128)
    deg8 = dout[:, :N, 0:8].reshape(2 * N, 8)

    h1p = _mm1(x, W1, deg8)
    acc1 = _conv_kernel(h1p, srcb, dstb)[:, :N, :].reshape(2 * N, D)
    h2p = _mid(acc1, h1p, deg8, W2, b1.reshape(1, D))
    acc2 = _conv_kernel(h2p, srcb, dstb)[:, :N, :].reshape(2 * N, D)

    seg = jnp.concatenate([batch_s, batch_t + G]).astype(jnp.int32)
    seg = seg.reshape(2 * N, 1)
    lab = labels_s.astype(jnp.int32).reshape(G, 1)

    tot, clf, dl = _tail(acc2, h2p, deg8, seg, lab, b2.reshape(1, D),
                         Wc1, bc1.reshape(1, C), Wc2, bc2.reshape(1, C),
                         Wd, bd.reshape(1, 1))
    return tot.reshape(()), clf.reshape(()), dl.reshape(())


# speedup vs baseline: 23.2490x; 1.0332x over previous
import functools

import jax
import jax.numpy as jnp
from jax import lax
from jax.experimental import pallas as pl
from jax.experimental.pallas import tpu as pltpu
from jax.experimental.pallas import tpu_sc as plsc

N = 10000
E = 320000
D = 128
G = 64
C = 16

NTILES = 16
EPT = 20480
NB = EPT // 128
NP = 10240
RPT = NP // NTILES
CB = 8
NCHK = NB // CB

def _deg_body(dstb_hbm, ones_hbm, out_hbm, dacc, didx, ones_v, zbuf,
              semi, ssa, ssb):
    ss = (ssa, ssb)
    c = lax.axis_index("c")
    s = lax.axis_index("s")
    w = c * NTILES + s

    def zrow(i, carry):
        for j in range(D // 16):
            zbuf[i, pl.ds(j * 16, 16)] = jnp.zeros((16,), jnp.float32)
        return carry

    lax.fori_loop(0, 128, zrow, 0)
    row0 = s * RPT
    for k in range(RPT // 128):
        pltpu.sync_copy(zbuf, dacc.at[pl.ds(row0 + k * 128, 128)])
    pltpu.sync_copy(ones_hbm, ones_v)
    plsc.subcore_barrier()

    pltpu.async_copy(dstb_hbm.at[w].at[pl.ds(0, CB)], didx.at[0], semi)
    for ch in range(NCHK):
        b = ch % 2
        pltpu.make_async_copy(
            dstb_hbm.at[w].at[pl.ds(ch * CB, CB)], didx.at[b], semi).wait()

        def fire(j, carry, dv=didx.at[b], sem=ss[b]):
            pltpu.async_copy(ones_v, dacc.at[dv.at[j]], sem, add=True)
            return carry

        lax.fori_loop(0, CB, fire, 0)

        if ch > 0:
            def drain(j, carry, dv=didx.at[1 - b], sem=ss[1 - b]):
                pltpu.make_async_copy(ones_v, dacc.at[dv.at[j]], sem).wait()
                return carry

            lax.fori_loop(0, CB, drain, 0)
        if ch + 1 < NCHK:
            pltpu.async_copy(dstb_hbm.at[w].at[pl.ds((ch + 1) * CB, CB)],
                             didx.at[1 - b], semi)

    bl = (NCHK - 1) % 2

    def draind(j, carry):
        pltpu.make_async_copy(ones_v, dacc.at[didx.at[bl].at[j]],
                              ss[bl]).wait()
        return carry

    lax.fori_loop(0, CB, draind, 0)
    plsc.subcore_barrier()
    for k in range(RPT // 128):
        r0 = row0 + k * 128
        pltpu.sync_copy(dacc.at[pl.ds(r0, 128)], zbuf)
        pltpu.sync_copy(zbuf, out_hbm.at[c].at[pl.ds(r0, 128)])


BW = 64
NBAT = EPT // BW
CCH = 16
NCC = NBAT // CCH


def _conv_body(h_hbm, srcb_hbm, dstb_hbm, out_hbm,
               acc, sidx, didx, msg, semi,
               gs0, gs1, gs2, gs3, ss0, ss1, ss2, ss3):
    gs = (gs0, gs1, gs2, gs3)
    ss = (ss0, ss1, ss2, ss3)
    c = lax.axis_index("c")
    s = lax.axis_index("s")
    w = c * NTILES + s

    def zrow(i, carry):
        for j in range(D // 16):
            msg[0, i, pl.ds(j * 16, 16)] = jnp.zeros((16,), jnp.float32)
        return carry

    lax.fori_loop(0, BW, zrow, 0)
    row0 = s * RPT
    for k in range(RPT // BW):
        pltpu.sync_copy(msg.at[0], acc.at[pl.ds(row0 + k * BW, BW)])
    plsc.subcore_barrier()

    pltpu.async_copy(srcb_hbm.at[w].at[pl.ds(0, CCH)], sidx.at[0], semi)
    pltpu.async_copy(dstb_hbm.at[w].at[pl.ds(0, CCH)], didx.at[0], semi)

    for ch in range(NCC):
        cb = ch % 2
        pltpu.make_async_copy(
            srcb_hbm.at[w].at[pl.ds(ch * CCH, CCH)], sidx.at[cb], semi).wait()
        pltpu.make_async_copy(
            dstb_hbm.at[w].at[pl.ds(ch * CCH, CCH)], didx.at[cb], semi).wait()
        if ch + 1 < NCC:
            o0 = (ch + 1) * CCH
            pltpu.async_copy(srcb_hbm.at[w].at[pl.ds(o0, CCH)],
                             sidx.at[1 - cb], semi)
            pltpu.async_copy(dstb_hbm.at[w].at[pl.ds(o0, CCH)],
                             didx.at[1 - cb], semi)
        sv = sidx.at[cb]
        dv = didx.at[cb]

        for p in range(3):
            if ch > 0:
                pltpu.make_async_copy(msg.at[p], acc.at[dv.at[p]],
                                      ss[p]).wait()
            pltpu.async_copy(h_hbm.at[sv.at[p]], msg.at[p], gs[p])

        def group(k2, carry, sv=sv, dv=dv, first=(ch == 0)):
            rg = 4 * k2
            for u in range(4):
                r = rg + u
                pltpu.make_async_copy(h_hbm.at[sv.at[r]], msg.at[u],
                                      gs[u]).wait()
                pltpu.async_copy(msg.at[u], acc.at[dv.at[r]], ss[u], add=True)
                bl = (u + 3) % 4
                if first:
                    @pl.when(jnp.logical_and(r + 3 < CCH, r >= 1))
                    def _(bl=bl, r=r):
                        pltpu.make_async_copy(msg.at[bl], acc.at[dv.at[r]],
                                              ss[bl]).wait()

                    @pl.when(r + 3 < CCH)
                    def _(bl=bl, r=r):
                        pltpu.async_copy(h_hbm.at[sv.at[r + 3]], msg.at[bl],
                                         gs[bl])
                else:
                    @pl.when(r + 3 < CCH)
                    def _(bl=bl, r=r):
                        pltpu.make_async_copy(msg.at[bl], acc.at[dv.at[r]],
                                              ss[bl]).wait()
                        pltpu.async_copy(h_hbm.at[sv.at[r + 3]], msg.at[bl],
                                         gs[bl])
            return carry

        lax.fori_loop(0, CCH // 4, group, 0)

    for p in range(4):
        pltpu.make_async_copy(msg.at[p], acc.at[didx.at[0].at[0]],
                              ss[p]).wait()
    plsc.subcore_barrier()

    for k in range(RPT // BW):
        r0 = row0 + k * BW
        pltpu.sync_copy(acc.at[pl.ds(r0, BW)], msg.at[0])
        pltpu.sync_copy(msg.at[0], out_hbm.at[c].at[pl.ds(r0, BW)])


@functools.lru_cache(maxsize=None)
def _sc_kernels():
    mesh = plsc.VectorSubcoreMesh(core_axis_name="c", subcore_axis_name="s")
    deg = pl.kernel(
        _deg_body,
        out_type=jax.ShapeDtypeStruct((2, NP, D), jnp.float32),
        mesh=mesh,
        scratch_types=[
            pltpu.VMEM_SHARED((NP, D), jnp.float32),
            pltpu.VMEM((2, CB, 128), jnp.int32),
            pltpu.VMEM((128, D), jnp.float32),
            pltpu.VMEM((128, D), jnp.float32),
            pltpu.SemaphoreType.DMA,
            pltpu.SemaphoreType.DMA,
            pltpu.SemaphoreType.DMA,
        ],
    )
    conv = pl.kernel(
        _conv_body,
        out_type=jax.ShapeDtypeStruct((2, NP, D), jnp.float32),
        mesh=mesh,
        scratch_types=[
            pltpu.VMEM_SHARED((NP, D), jnp.float32),
            pltpu.VMEM((2, CCH, BW), jnp.int32),
            pltpu.VMEM((2, CCH, BW), jnp.int32),
            pltpu.VMEM((4, BW, D), jnp.float32),
        ] + [pltpu.SemaphoreType.DMA] * 9,
    )
    return deg, conv


def _dinv(deg8):
    deg = deg8[:, 0:1] + 1.0
    return lax.rsqrt(jnp.maximum(deg, 1e-12))


def _mm1_body(x_ref, w_ref, deg_ref, o_ref):
    h = jnp.dot(x_ref[...], w_ref[...], preferred_element_type=jnp.float32)
    o_ref[...] = h * _dinv(deg_ref[...])


def _mid_body(acc_ref, h1p_ref, deg_ref, w2_ref, b1_ref, o_ref):
    dinv = _dinv(deg_ref[...])
    z = jnp.maximum(dinv * (acc_ref[...] + h1p_ref[...]) + b1_ref[...], 0.0)
    o_ref[...] = jnp.dot(z, w2_ref[...], preferred_element_type=jnp.float32) * dinv


def _tail_body(acc_ref, h2p_ref, deg_ref, seg_ref, lab_ref, b2_ref,
               wc1_ref, bc1_ref, wc2_ref, bc2_ref, wd_ref, bd_ref,
               o1_ref, o2_ref, o3_ref):
    dinv = _dinv(deg_ref[...])
    x = dinv * (acc_ref[...] + h2p_ref[...]) + b2_ref[...]
    seg = seg_ref[...]
    oh = (seg == lax.broadcasted_iota(jnp.int32, (2 * N, 2 * G), 1))
    oh = oh.astype(jnp.float32)
    cdims = (((0,), (0,)), ((), ()))
    pooled = lax.dot_general(oh, x, cdims,
                             preferred_element_type=jnp.float32)
    cnt = lax.dot_general(oh, jnp.ones((2 * N, 2 * G), jnp.float32), cdims,
                          preferred_element_type=jnp.float32)
    xb = pooled / jnp.maximum(cnt, 1.0)
    xds = xb[0:G, :]
    xdt = xb[G:2 * G, :]
    hh = jnp.maximum(
        jnp.dot(xds, wc1_ref[...], preferred_element_type=jnp.float32)
        + bc1_ref[...], 0.0)
    p = jax.nn.sigmoid(
        jnp.dot(hh, wc2_ref[...], preferred_element_type=jnp.float32)
        + bc2_ref[...])
    p = jnp.clip(p, 1e-7, 1.0 - 1e-7)
    y = (lab_ref[...] == lax.broadcasted_iota(jnp.int32, (G, C), 1))
    y = y.astype(jnp.float32)
    clf = -jnp.mean(y * jnp.log(p) + (1.0 - y) * jnp.log(1.0 - p))
    cr = jax.nn.sigmoid(
        jnp.dot(xb, wd_ref[...], preferred_element_type=jnp.float32)
        + bd_ref[...])
    m_ds = jnp.mean(cr[0:G, :])
    m_dt = jnp.mean(cr[G:2 * G, :])
    dl = jnp.abs(m_ds - m_dt)
    o1_ref[...] = (clf + dl).reshape(1, 1)
    o2_ref[...] = clf.reshape(1, 1)
    o3_ref[...] = dl.reshape(1, 1)


_mm1 = pl.pallas_call(
    _mm1_body,
    out_shape=jax.ShapeDtypeStruct((2 * N, D), jnp.float32),
)

_mid = pl.pallas_call(
    _mid_body,
    out_shape=jax.ShapeDtypeStruct((2 * N, D), jnp.float32),
)

_tail = pl.pallas_call(
    _tail_body,
    out_shape=(
        jax.ShapeDtypeStruct((1, 1), jnp.float32),
        jax.ShapeDtypeStruct((1, 1), jnp.float32),
        jax.ShapeDtypeStruct((1, 1), jnp.float32),
    ),
)


def _edge_blocks(src, dst, goff):
    pad = EPT * NTILES - E
    ar = jnp.arange(pad, dtype=jnp.int32)
    pad_src = (ar * 67) % N + goff
    pad_dst = N + (ar % 128)
    s_all = jnp.concatenate([src + goff, pad_src]).reshape(NTILES, EPT)
    d_all = jnp.concatenate([dst, pad_dst]).reshape(NTILES, EPT)
    return s_all, d_all


def kernel(features_s, edge_index_s, batch_s, labels_s,
           features_t, edge_index_t, batch_t,
           W1, b1, W2, b2, Wc1, bc1, Wc2, bc2, Wd, bd):
    x = jnp.concatenate([features_s, features_t], axis=0)

    ss, ds_ = _edge_blocks(edge_index_s[0], edge_index_s[1], 0)
    st, dt_ = _edge_blocks(edge_index_t[0], edge_index_t[1], N)
    srcb = jnp.concatenate([ss[None], st[None]]).reshape(2 * NTILES, EPT)
    dstb = jnp.concatenate([ds_[None], dt_[None]]).reshape(2 * NTILES, EPT)
    srcb_c = srcb.reshape(2 * NTILES, NBAT, BW)
    dstb_c = dstb.reshape(2 * NTILES, NBAT, BW)
    dstb_d = dstb.reshape(2 * NTILES, NB, 128)

    _deg_kernel, _conv_kernel = _sc_kernels()
    ones128 = jnp.ones((128, D), jnp.float32)
    dout = _deg_kernel(dstb_d, ones128)
    deg8 = dout[:, :N, 0:8].reshape(2 * N, 8)

    h1p = _mm1(x, W1, deg8)
    acc1 = _conv_kernel(h1p, srcb_c, dstb_c)[:, :N, :].reshape(2 * N, D)
    h2p = _mid(acc1, h1p, deg8, W2, b1.reshape(1, D))
    acc2 = _conv_kernel(h2p, srcb_c, dstb_c)[:, :N, :].reshape(2 * N, D)

    seg = jnp.concatenate([batch_s, batch_t + G]).astype(jnp.int32)
    seg = seg.reshape(2 * N, 1)
    lab = labels_s.astype(jnp.int32).reshape(G, 1)

    tot, clf, dl = _tail(acc2, h2p, deg8, seg, lab, b2.reshape(1, D),
                         Wc1, bc1.reshape(1, C), Wc2, bc2.reshape(1, C),
                         Wd, bd.reshape(1, 1))
    return tot.reshape(()), clf.reshape(()), dl.reshape(())


# speedup vs baseline: 27.7268x; 1.1926x over previous
import functools

import jax
import jax.numpy as jnp
from jax import lax
from jax.experimental import pallas as pl
from jax.experimental.pallas import tpu as pltpu
from jax.experimental.pallas import tpu_sc as plsc

N = 10000
E = 320000
D = 128
G = 64
C = 16

NTILES = 16
EPT = 20480
NP = 10240
RPT = NP // NTILES

def _deg_body(dstf_hbm, out_hbm, shared, hist, idxv, tmpv, accv):
    c = lax.axis_index("c")
    s = lax.axis_index("s")
    w = c * NTILES + s

    def z(i, carry):
        hist[pl.ds(i * 16, 16)] = jnp.zeros((16,), jnp.float32)
        return carry

    lax.fori_loop(0, NP // 16, z, 0)
    pltpu.sync_copy(dstf_hbm.at[w], idxv)
    ones16 = jnp.ones((16,), jnp.float32)

    def step(i, carry):
        k16 = idxv[pl.ds(i * 16, 16)]
        plsc.addupdate_scatter(hist, [k16], ones16)
        return carry

    lax.fori_loop(0, EPT // 16, step, 0)
    pltpu.sync_copy(hist, shared.at[s])
    plsc.subcore_barrier()

    r0 = s * RPT

    def zacc(i, carry):
        accv[pl.ds(i * 16, 16)] = jnp.zeros((16,), jnp.float32)
        return carry

    lax.fori_loop(0, RPT // 16, zacc, 0)
    for t in range(NTILES):
        pltpu.sync_copy(shared.at[t].at[pl.ds(r0, RPT)], tmpv)

        def addt(i, carry):
            accv[pl.ds(i * 16, 16)] = (accv[pl.ds(i * 16, 16)]
                                       + tmpv[pl.ds(i * 16, 16)])
            return carry

        lax.fori_loop(0, RPT // 16, addt, 0)
    pltpu.sync_copy(accv, out_hbm.at[c].at[pl.ds(r0, RPT)])


BW = 64
NBAT = EPT // BW
CCH = 16
NCC = NBAT // CCH


def _conv_body(h_hbm, srcb_hbm, dstb_hbm, out_hbm,
               acc, sidx, didx, msg, semi,
               gs0, gs1, gs2, gs3, ss0, ss1, ss2, ss3):
    gs = (gs0, gs1, gs2, gs3)
    ss = (ss0, ss1, ss2, ss3)
    c = lax.axis_index("c")
    s = lax.axis_index("s")
    w = c * NTILES + s

    def zrow(i, carry):
        for j in range(D // 16):
            msg[0, i, pl.ds(j * 16, 16)] = jnp.zeros((16,), jnp.float32)
        return carry

    lax.fori_loop(0, BW, zrow, 0)
    row0 = s * RPT
    for k in range(RPT // BW):
        pltpu.sync_copy(msg.at[0], acc.at[pl.ds(row0 + k * BW, BW)])
    plsc.subcore_barrier()

    pltpu.async_copy(srcb_hbm.at[w].at[pl.ds(0, CCH)], sidx.at[0], semi)
    pltpu.async_copy(dstb_hbm.at[w].at[pl.ds(0, CCH)], didx.at[0], semi)

    for ch in range(NCC):
        cb = ch % 2
        pltpu.make_async_copy(
            srcb_hbm.at[w].at[pl.ds(ch * CCH, CCH)], sidx.at[cb], semi).wait()
        pltpu.make_async_copy(
            dstb_hbm.at[w].at[pl.ds(ch * CCH, CCH)], didx.at[cb], semi).wait()
        if ch + 1 < NCC:
            o0 = (ch + 1) * CCH
            pltpu.async_copy(srcb_hbm.at[w].at[pl.ds(o0, CCH)],
                             sidx.at[1 - cb], semi)
            pltpu.async_copy(dstb_hbm.at[w].at[pl.ds(o0, CCH)],
                             didx.at[1 - cb], semi)
        sv = sidx.at[cb]
        dv = didx.at[cb]

        for p in range(3):
            if ch > 0:
                pltpu.make_async_copy(msg.at[p], acc.at[dv.at[p]],
                                      ss[p]).wait()
            pltpu.async_copy(h_hbm.at[sv.at[p]], msg.at[p], gs[p])

        def group(k2, carry, sv=sv, dv=dv, first=(ch == 0)):
            rg = 4 * k2
            for u in range(4):
                r = rg + u
                pltpu.make_async_copy(h_hbm.at[sv.at[r]], msg.at[u],
                                      gs[u]).wait()
                pltpu.async_copy(msg.at[u], acc.at[dv.at[r]], ss[u], add=True)
                bl = (u + 3) % 4
                if first:
                    @pl.when(jnp.logical_and(r + 3 < CCH, r >= 1))
                    def _(bl=bl, r=r):
                        pltpu.make_async_copy(msg.at[bl], acc.at[dv.at[r]],
                                              ss[bl]).wait()

                    @pl.when(r + 3 < CCH)
                    def _(bl=bl, r=r):
                        pltpu.async_copy(h_hbm.at[sv.at[r + 3]], msg.at[bl],
                                         gs[bl])
                else:
                    @pl.when(r + 3 < CCH)
                    def _(bl=bl, r=r):
                        pltpu.make_async_copy(msg.at[bl], acc.at[dv.at[r]],
                                              ss[bl]).wait()
                        pltpu.async_copy(h_hbm.at[sv.at[r + 3]], msg.at[bl],
                                         gs[bl])
            return carry

        lax.fori_loop(0, CCH // 4, group, 0)

    for p in range(4):
        pltpu.make_async_copy(msg.at[p], acc.at[didx.at[0].at[0]],
                              ss[p]).wait()
    plsc.subcore_barrier()

    for k in range(RPT // BW):
        r0 = row0 + k * BW
        pltpu.sync_copy(acc.at[pl.ds(r0, BW)], msg.at[0])
        pltpu.sync_copy(msg.at[0], out_hbm.at[c].at[pl.ds(r0, BW)])


@functools.lru_cache(maxsize=None)
def _sc_kernels():
    mesh = plsc.VectorSubcoreMesh(core_axis_name="c", subcore_axis_name="s")
    deg = pl.kernel(
        _deg_body,
        out_type=jax.ShapeDtypeStruct((2, NP), jnp.float32),
        mesh=mesh,
        scratch_types=[
            pltpu.VMEM_SHARED((NTILES, NP), jnp.float32),
            pltpu.VMEM((NP,), jnp.float32),
            pltpu.VMEM((EPT,), jnp.int32),
            pltpu.VMEM((RPT,), jnp.float32),
            pltpu.VMEM((RPT,), jnp.float32),
        ],
        compiler_params=pltpu.CompilerParams(needs_layout_passes=False),
    )
    conv = pl.kernel(
        _conv_body,
        out_type=jax.ShapeDtypeStruct((2, NP, D), jnp.float32),
        mesh=mesh,
        scratch_types=[
            pltpu.VMEM_SHARED((NP, D), jnp.float32),
            pltpu.VMEM((2, CCH, BW), jnp.int32),
            pltpu.VMEM((2, CCH, BW), jnp.int32),
            pltpu.VMEM((4, BW, D), jnp.float32),
        ] + [pltpu.SemaphoreType.DMA] * 9,
    )
    return deg, conv


def _dinv(deg8):
    deg = deg8[:, 0:1] + 1.0
    return lax.rsqrt(jnp.maximum(deg, 1e-12))


def _mm1_body(x_ref, w_ref, deg_ref, o_ref):
    h = jnp.dot(x_ref[...], w_ref[...], preferred_element_type=jnp.float32)
    o_ref[...] = h * _dinv(deg_ref[...])


def _mid_body(acc_ref, h1p_ref, deg_ref, w2_ref, b1_ref, o_ref):
    dinv = _dinv(deg_ref[...])
    z = jnp.maximum(dinv * (acc_ref[...] + h1p_ref[...]) + b1_ref[...], 0.0)
    o_ref[...] = jnp.dot(z, w2_ref[...], preferred_element_type=jnp.float32) * dinv


def _tail_body(acc_ref, h2p_ref, deg_ref, seg_ref, lab_ref, b2_ref,
               wc1_ref, bc1_ref, wc2_ref, bc2_ref, wd_ref, bd_ref,
               o1_ref, o2_ref, o3_ref):
    dinv = _dinv(deg_ref[...])
    x = dinv * (acc_ref[...] + h2p_ref[...]) + b2_ref[...]
    seg = seg_ref[...]
    oh = (seg == lax.broadcasted_iota(jnp.int32, (2 * N, 2 * G), 1))
    oh = oh.astype(jnp.float32)
    cdims = (((0,), (0,)), ((), ()))
    pooled = lax.dot_general(oh, x, cdims,
                             preferred_element_type=jnp.float32)
    cnt = lax.dot_general(oh, jnp.ones((2 * N, 2 * G), jnp.float32), cdims,
                          preferred_element_type=jnp.float32)
    xb = pooled / jnp.maximum(cnt, 1.0)
    xds = xb[0:G, :]
    xdt = xb[G:2 * G, :]
    hh = jnp.maximum(
        jnp.dot(xds, wc1_ref[...], preferred_element_type=jnp.float32)
        + bc1_ref[...], 0.0)
    p = jax.nn.sigmoid(
        jnp.dot(hh, wc2_ref[...], preferred_element_type=jnp.float32)
        + bc2_ref[...])
    p = jnp.clip(p, 1e-7, 1.0 - 1e-7)
    y = (lab_ref[...] == lax.broadcasted_iota(jnp.int32, (G, C), 1))
    y = y.astype(jnp.float32)
    clf = -jnp.mean(y * jnp.log(p) + (1.0 - y) * jnp.log(1.0 - p))
    cr = jax.nn.sigmoid(
        jnp.dot(xb, wd_ref[...], preferred_element_type=jnp.float32)
        + bd_ref[...])
    m_ds = jnp.mean(cr[0:G, :])
    m_dt = jnp.mean(cr[G:2 * G, :])
    dl = jnp.abs(m_ds - m_dt)
    o1_ref[...] = (clf + dl).reshape(1, 1)
    o2_ref[...] = clf.reshape(1, 1)
    o3_ref[...] = dl.reshape(1, 1)


_mm1 = pl.pallas_call(
    _mm1_body,
    out_shape=jax.ShapeDtypeStruct((2 * N, D), jnp.float32),
)

_mid = pl.pallas_call(
    _mid_body,
    out_shape=jax.ShapeDtypeStruct((2 * N, D), jnp.float32),
)

_tail = pl.pallas_call(
    _tail_body,
    out_shape=(
        jax.ShapeDtypeStruct((1, 1), jnp.float32),
        jax.ShapeDtypeStruct((1, 1), jnp.float32),
        jax.ShapeDtypeStruct((1, 1), jnp.float32),
    ),
)


def _edge_blocks(src, dst, goff):
    pad = EPT * NTILES - E
    ar = jnp.arange(pad, dtype=jnp.int32)
    pad_src = (ar * 67) % N + goff
    pad_dst = N + (ar % 128)
    s_all = jnp.concatenate([src + goff, pad_src]).reshape(NTILES, EPT)
    d_all = jnp.concatenate([dst, pad_dst]).reshape(NTILES, EPT)
    return s_all, d_all


def kernel(features_s, edge_index_s, batch_s, labels_s,
           features_t, edge_index_t, batch_t,
           W1, b1, W2, b2, Wc1, bc1, Wc2, bc2, Wd, bd):
    x = jnp.concatenate([features_s, features_t], axis=0)

    ss, ds_ = _edge_blocks(edge_index_s[0], edge_index_s[1], 0)
    st, dt_ = _edge_blocks(edge_index_t[0], edge_index_t[1], N)
    srcb = jnp.concatenate([ss[None], st[None]]).reshape(2 * NTILES, EPT)
    dstb = jnp.concatenate([ds_[None], dt_[None]]).reshape(2 * NTILES, EPT)
    srcb_c = srcb.reshape(2 * NTILES, NBAT, BW)
    dstb_c = dstb.reshape(2 * NTILES, NBAT, BW)

    _deg_kernel, _conv_kernel = _sc_kernels()
    dout = _deg_kernel(dstb)
    deg8 = dout[:, :N].reshape(2 * N, 1)

    h1p = _mm1(x, W1, deg8)
    acc1 = _conv_kernel(h1p, srcb_c, dstb_c)[:, :N, :].reshape(2 * N, D)
    h2p = _mid(acc1, h1p, deg8, W2, b1.reshape(1, D))
    acc2 = _conv_kernel(h2p, srcb_c, dstb_c)[:, :N, :].reshape(2 * N, D)

    seg = jnp.concatenate([batch_s, batch_t + G]).astype(jnp.int32)
    seg = seg.reshape(2 * N, 1)
    lab = labels_s.astype(jnp.int32).reshape(G, 1)

    tot, clf, dl = _tail(acc2, h2p, deg8, seg, lab, b2.reshape(1, D),
                         Wc1, bc1.reshape(1, C), Wc2, bc2.reshape(1, C),
                         Wd, bd.reshape(1, 1))
    return tot.reshape(()), clf.reshape(()), dl.reshape(())
